# baseline (speedup 1.0000x reference)
import functools

import jax
import jax.numpy as jnp
from jax import lax
from jax.experimental import pallas as pl
from jax.experimental.pallas import tpu as pltpu
from jax.experimental.pallas import tpu_sc as plsc

N_BATCH = 4
N_ROW = 4096
N_COL = 2048
NC = 2
NS = 16
NW = NC * NS
L = 16
CB = 128
NCB = N_COL // CB
G = CB // L
TASKS_W = N_BATCH * NCB // NW
R = 256
N_CHUNK = N_ROW // R


def _sc_argmin(x_hbm, o_hbm, buf0, buf1, obuf, sem0, sem1):
    c = lax.axis_index("c")
    s = lax.axis_index("s")
    wid = s * NC + c
    bufs = (buf0, buf1)
    sems = (sem0, sem1)

    for t in range(TASKS_W):
        task = wid * TASKS_W + t
        b = task // NCB
        col0 = pl.multiple_of((task % NCB) * CB, CB)

        def chunk_src(k, _b=b, _col0=col0):
            return x_hbm.at[_b, pl.ds(k * R, R), pl.ds(_col0, CB)]

        pltpu.make_async_copy(chunk_src(0), bufs[0], sems[0]).start()
        mns = [jnp.full((L,), jnp.inf, jnp.float32) for _ in range(G)]
        mis = [jnp.full((L,), 0, jnp.int32) for _ in range(G)]
        for k in range(N_CHUNK):
            buf = bufs[k % 2]
            if k + 1 < N_CHUNK:
                pltpu.make_async_copy(
                    chunk_src(k + 1), bufs[(k + 1) % 2], sems[(k + 1) % 2]
                ).start()
            pltpu.make_async_copy(chunk_src(k), buf, sems[k % 2]).wait()

            def row_body(r, carry, _buf=buf, _base=k * R):
                c_mns, c_mis = carry
                ridx = jnp.full((L,), 0, jnp.int32) + (_base + r)
                n_mns, n_mis = [], []
                for g in range(G):
                    v = _buf[r, pl.ds(g * L, L)]
                    better = v < c_mns[g]
                    n_mns.append(jnp.where(better, v, c_mns[g]))
                    n_mis.append(jnp.where(better, ridx, c_mis[g]))
                return tuple(n_mns), tuple(n_mis)

            mns, mis = lax.fori_loop(
                0, R, row_body, (tuple(mns), tuple(mis)), unroll=8
            )
            mns, mis = list(mns), list(mis)

        for g in range(G):
            obuf[pl.ds(g * L, L)] = mis[g]
        pltpu.sync_copy(obuf, o_hbm.at[b, pl.ds(col0, CB)])


def _sc_call(x):
    mesh = plsc.VectorSubcoreMesh(core_axis_name="c", subcore_axis_name="s")
    kfn = functools.partial(
        pl.kernel,
        mesh=mesh,
        out_type=jax.ShapeDtypeStruct((N_BATCH, N_COL), jnp.int32),
        scratch_types=[
            pltpu.VMEM((R, CB), jnp.float32),
            pltpu.VMEM((R, CB), jnp.float32),
            pltpu.VMEM((CB,), jnp.int32),
            pltpu.SemaphoreType.DMA,
            pltpu.SemaphoreType.DMA,
        ],
    )(_sc_argmin)
    return kfn(x)


def kernel(x):
    return _sc_call(x).astype(jnp.int64)


# speedup vs baseline: 1.0935x; 1.0935x over previous
import functools

import jax
import jax.numpy as jnp
from jax import lax
from jax.experimental import pallas as pl
from jax.experimental.pallas import tpu as pltpu
from jax.experimental.pallas import tpu_sc as plsc

N_BATCH = 4
N_ROW = 4096
N_COL = 2048
NC = 2
NS = 16
NW = NC * NS
L = 16
CB = 128
NCB = N_COL // CB
G = CB // L
TASKS_W = N_BATCH * NCB // NW
R = 256
N_CHUNK = N_ROW // R


def _sc_argmin(x_hbm, o_hbm, buf0, buf1, obuf, sem0, sem1):
    c = lax.axis_index("c")
    s = lax.axis_index("s")
    wid = s * NC + c
    bufs = (buf0, buf1)
    sems = (sem0, sem1)

    for t in range(TASKS_W):
        task = wid * TASKS_W + t
        b = task // NCB
        col0 = pl.multiple_of((task % NCB) * CB, CB)

        def chunk_src(k, _b=b, _col0=col0):
            return x_hbm.at[_b, pl.ds(k * R, R), pl.ds(_col0, CB)]

        pltpu.make_async_copy(chunk_src(0), bufs[0], sems[0]).start()
        mns = [jnp.full((L,), jnp.inf, jnp.float32) for _ in range(G)]
        mis = [jnp.full((L,), 0, jnp.int32) for _ in range(G)]
        for k in range(N_CHUNK):
            buf = bufs[k % 2]
            if k + 1 < N_CHUNK:
                pltpu.make_async_copy(
                    chunk_src(k + 1), bufs[(k + 1) % 2], sems[(k + 1) % 2]
                ).start()
            pltpu.make_async_copy(chunk_src(k), buf, sems[k % 2]).wait()

            def row_body(r, carry, _buf=buf, _base=k * R):
                c_mns, c_mis = carry
                ridx = jnp.full((L,), 0, jnp.int32) + (_base + r)
                n_mns, n_mis = [], []
                for g in range(G):
                    v = _buf[r, pl.ds(g * L, L)]
                    better = v < c_mns[g]
                    n_mns.append(jnp.where(better, v, c_mns[g]))
                    n_mis.append(jnp.where(better, ridx, c_mis[g]))
                return tuple(n_mns), tuple(n_mis)

            mns, mis = lax.fori_loop(
                0, R, row_body, (tuple(mns), tuple(mis)), unroll=4
            )
            mns, mis = list(mns), list(mis)

        for g in range(G):
            obuf[pl.ds(g * L, L)] = mis[g]
        pltpu.sync_copy(obuf, o_hbm.at[b, pl.ds(col0, CB)])


def _sc_call(x):
    mesh = plsc.VectorSubcoreMesh(core_axis_name="c", subcore_axis_name="s")
    kfn = functools.partial(
        pl.kernel,
        mesh=mesh,
        out_type=jax.ShapeDtypeStruct((N_BATCH, N_COL), jnp.int32),
        scratch_types=[
            pltpu.VMEM((R, CB), jnp.float32),
            pltpu.VMEM((R, CB), jnp.float32),
            pltpu.VMEM((CB,), jnp.int32),
            pltpu.SemaphoreType.DMA,
            pltpu.SemaphoreType.DMA,
        ],
    )(_sc_argmin)
    return kfn(x)


def kernel(x):
    return _sc_call(x).astype(jnp.int64)


# speedup vs baseline: 1.3911x; 1.2722x over previous
import functools

import jax
import jax.numpy as jnp
from jax import lax
from jax.experimental import pallas as pl
from jax.experimental.pallas import tpu as pltpu
from jax.experimental.pallas import tpu_sc as plsc

N_BATCH = 4
N_ROW = 4096
N_COL = 2048
NC = 2
NS = 16
NW = NC * NS
L = 16
CB = 128
NCB = N_COL // CB
G = CB // L
TASKS_W = N_BATCH * NCB // NW
R = 256
N_CHUNK = N_ROW // R


def _sc_argmin(x_hbm, o_hbm, buf0, buf1, obuf, sem0, sem1):
    c = lax.axis_index("c")
    s = lax.axis_index("s")
    wid = s * NC + c
    bufs = (buf0, buf1)
    sems = (sem0, sem1)

    for t in range(TASKS_W):
        task = wid * TASKS_W + t
        b = task // NCB
        col0 = pl.multiple_of((task % NCB) * CB, CB)

        def chunk_src(k, _b=b, _col0=col0):
            return x_hbm.at[_b, pl.ds(k * R, R), pl.ds(_col0, CB)]

        pltpu.make_async_copy(chunk_src(0), bufs[0], sems[0]).start()
        mns = [jnp.full((L,), jnp.inf, jnp.float32) for _ in range(G)]
        mis = [jnp.full((L,), 0, jnp.int32) for _ in range(G)]
        for k in range(N_CHUNK):
            buf = bufs[k % 2]
            if k + 1 < N_CHUNK:
                pltpu.make_async_copy(
                    chunk_src(k + 1), bufs[(k + 1) % 2], sems[(k + 1) % 2]
                ).start()
            pltpu.make_async_copy(chunk_src(k), buf, sems[k % 2]).wait()

            def row_body(r2, carry, _buf=buf, _base=k * R):
                c_mns, c_mis = carry
                r0 = r2 * 2
                i0 = jnp.full((L,), 0, jnp.int32) + (_base + r0)
                i1 = i0 + 1
                n_mns, n_mis = [], []
                for g in range(G):
                    v0 = _buf[r0, pl.ds(g * L, L)]
                    v1 = _buf[r0 + 1, pl.ds(g * L, L)]
                    c01 = v1 < v0
                    m01 = jnp.minimum(v0, v1)
                    i01 = jnp.where(c01, i1, i0)
                    b = m01 < c_mns[g]
                    n_mns.append(jnp.where(b, m01, c_mns[g]))
                    n_mis.append(jnp.where(b, i01, c_mis[g]))
                return tuple(n_mns), tuple(n_mis)

            mns, mis = lax.fori_loop(
                0, R // 2, row_body, (tuple(mns), tuple(mis)), unroll=2
            )
            mns, mis = list(mns), list(mis)

        for g in range(G):
            obuf[pl.ds(g * L, L)] = mis[g]
        pltpu.sync_copy(obuf, o_hbm.at[b, pl.ds(col0, CB)])


def _sc_call(x):
    mesh = plsc.VectorSubcoreMesh(core_axis_name="c", subcore_axis_name="s")
    kfn = functools.partial(
        pl.kernel,
        mesh=mesh,
        out_type=jax.ShapeDtypeStruct((N_BATCH, N_COL), jnp.int32),
        scratch_types=[
            pltpu.VMEM((R, CB), jnp.float32),
            pltpu.VMEM((R, CB), jnp.float32),
            pltpu.VMEM((CB,), jnp.int32),
            pltpu.SemaphoreType.DMA,
            pltpu.SemaphoreType.DMA,
        ],
    )(_sc_argmin)
    return kfn(x)


def kernel(x):
    return _sc_call(x).astype(jnp.int64)


# speedup vs baseline: 1.4313x; 1.0289x over previous
import functools

import jax
import jax.numpy as jnp
from jax import lax
from jax.experimental import pallas as pl
from jax.experimental.pallas import tpu as pltpu
from jax.experimental.pallas import tpu_sc as plsc

N_BATCH = 4
N_ROW = 4096
N_COL = 2048
NC = 2
NS = 16
NW = NC * NS
L = 16
CB = 128
NCB = N_COL // CB
G = CB // L
TASKS_W = N_BATCH * NCB // NW
R = 256
N_CHUNK = N_ROW // R


def _sc_argmin(x_hbm, o_hbm, buf0, buf1, obuf, sem0, sem1):
    c = lax.axis_index("c")
    s = lax.axis_index("s")
    wid = s * NC + c
    bufs = (buf0, buf1)
    sems = (sem0, sem1)

    for t in range(TASKS_W):
        task = wid * TASKS_W + t
        b = task // NCB
        col0 = pl.multiple_of((task % NCB) * CB, CB)

        def chunk_src(k, _b=b, _col0=col0):
            return x_hbm.at[_b, pl.ds(k * R, R), pl.ds(_col0, CB)]

        pltpu.make_async_copy(chunk_src(0), bufs[0], sems[0]).start()
        mns = [jnp.full((L,), jnp.inf, jnp.float32) for _ in range(G)]
        mis = [jnp.full((L,), 0, jnp.int32) for _ in range(G)]
        for k in range(N_CHUNK):
            buf = bufs[k % 2]
            if k + 1 < N_CHUNK:
                pltpu.make_async_copy(
                    chunk_src(k + 1), bufs[(k + 1) % 2], sems[(k + 1) % 2]
                ).start()
            pltpu.make_async_copy(chunk_src(k), buf, sems[k % 2]).wait()

            def row_body(r4, carry, _buf=buf, _base=k * R):
                c_mns, c_mis = carry
                r0 = r4 * 4
                i0 = jnp.full((L,), 0, jnp.int32) + (_base + r0)
                i1 = i0 + 1
                i2 = i0 + 2
                i3 = i0 + 3
                n_mns, n_mis = [], []
                for g in range(G):
                    v0 = _buf[r0, pl.ds(g * L, L)]
                    v1 = _buf[r0 + 1, pl.ds(g * L, L)]
                    v2 = _buf[r0 + 2, pl.ds(g * L, L)]
                    v3 = _buf[r0 + 3, pl.ds(g * L, L)]
                    c01 = v1 < v0
                    m01 = jnp.minimum(v0, v1)
                    i01 = jnp.where(c01, i1, i0)
                    c23 = v3 < v2
                    m23 = jnp.minimum(v2, v3)
                    i23 = jnp.where(c23, i3, i2)
                    c03 = m23 < m01
                    m03 = jnp.minimum(m01, m23)
                    i03 = jnp.where(c03, i23, i01)
                    b = m03 < c_mns[g]
                    n_mns.append(jnp.where(b, m03, c_mns[g]))
                    n_mis.append(jnp.where(b, i03, c_mis[g]))
                return tuple(n_mns), tuple(n_mis)

            mns, mis = lax.fori_loop(
                0, R // 4, row_body, (tuple(mns), tuple(mis)), unroll=1
            )
            mns, mis = list(mns), list(mis)

        for g in range(G):
            obuf[pl.ds(g * L, L)] = mis[g]
        pltpu.sync_copy(obuf, o_hbm.at[b, pl.ds(col0, CB)])


def _sc_call(x):
    mesh = plsc.VectorSubcoreMesh(core_axis_name="c", subcore_axis_name="s")
    kfn = functools.partial(
        pl.kernel,
        mesh=mesh,
        out_type=jax.ShapeDtypeStruct((N_BATCH, N_COL), jnp.int32),
        scratch_types=[
            pltpu.VMEM((R, CB), jnp.float32),
            pltpu.VMEM((R, CB), jnp.float32),
            pltpu.VMEM((CB,), jnp.int32),
            pltpu.SemaphoreType.DMA,
            pltpu.SemaphoreType.DMA,
        ],
    )(_sc_argmin)
    return kfn(x)


def kernel(x):
    return _sc_call(x).astype(jnp.int64)


# speedup vs baseline: 1.8888x; 1.3196x over previous
import functools

import jax
import jax.numpy as jnp
from jax import lax
from jax.experimental import pallas as pl
from jax.experimental.pallas import tpu as pltpu
from jax.experimental.pallas import tpu_sc as plsc

N_BATCH = 4
N_ROW = 4096
N_COL = 2048
NC = 2
NS = 16
NW = NC * NS
L = 16
CB = 128
NCB = N_COL // CB
G = CB // L
NCB_SC = 8
N_COL_SC = NCB_SC * CB
TASKS_W = N_BATCH * NCB_SC // NW
R = 256
N_CHUNK = N_ROW // R


def _sc_argmin(x_hbm, o_hbm, buf0, buf1, obuf, sem0, sem1):
    c = lax.axis_index("c")
    s = lax.axis_index("s")
    wid = s * NC + c
    bufs = (buf0, buf1)
    sems = (sem0, sem1)

    for t in range(TASKS_W):
        task = wid * TASKS_W + t
        b = task // NCB_SC
        col0 = pl.multiple_of((task % NCB_SC) * CB, CB)

        def chunk_src(k, _b=b, _col0=col0):
            return x_hbm.at[_b, pl.ds(k * R, R), pl.ds(_col0, CB)]

        pltpu.make_async_copy(chunk_src(0), bufs[0], sems[0]).start()
        mns = [jnp.full((L,), jnp.inf, jnp.float32) for _ in range(G)]
        mis = [jnp.full((L,), 0, jnp.int32) for _ in range(G)]
        for k in range(N_CHUNK):
            buf = bufs[k % 2]
            if k + 1 < N_CHUNK:
                pltpu.make_async_copy(
                    chunk_src(k + 1), bufs[(k + 1) % 2], sems[(k + 1) % 2]
                ).start()
            pltpu.make_async_copy(chunk_src(k), buf, sems[k % 2]).wait()

            def row_body(r4, carry, _buf=buf, _base=k * R):
                c_mns, c_mis = carry
                r0 = r4 * 4
                i0 = jnp.full((L,), 0, jnp.int32) + (_base + r0)
                i1 = i0 + 1
                i2 = i0 + 2
                i3 = i0 + 3
                n_mns, n_mis = [], []
                for g in range(G):
                    v0 = _buf[r0, pl.ds(g * L, L)]
                    v1 = _buf[r0 + 1, pl.ds(g * L, L)]
                    v2 = _buf[r0 + 2, pl.ds(g * L, L)]
                    v3 = _buf[r0 + 3, pl.ds(g * L, L)]
                    c01 = v1 < v0
                    m01 = jnp.minimum(v0, v1)
                    i01 = jnp.where(c01, i1, i0)
                    c23 = v3 < v2
                    m23 = jnp.minimum(v2, v3)
                    i23 = jnp.where(c23, i3, i2)
                    c03 = m23 < m01
                    m03 = jnp.minimum(m01, m23)
                    i03 = jnp.where(c03, i23, i01)
                    b = m03 < c_mns[g]
                    n_mns.append(jnp.where(b, m03, c_mns[g]))
                    n_mis.append(jnp.where(b, i03, c_mis[g]))
                return tuple(n_mns), tuple(n_mis)

            mns, mis = lax.fori_loop(
                0, R // 4, row_body, (tuple(mns), tuple(mis)), unroll=1
            )
            mns, mis = list(mns), list(mis)

        for g in range(G):
            obuf[pl.ds(g * L, L)] = mis[g]
        pltpu.sync_copy(obuf, o_hbm.at[b, pl.ds(col0, CB)])


def _sc_call(x):
    mesh = plsc.VectorSubcoreMesh(core_axis_name="c", subcore_axis_name="s")
    kfn = functools.partial(
        pl.kernel,
        mesh=mesh,
        out_type=jax.ShapeDtypeStruct((N_BATCH, N_COL_SC), jnp.int32),
        scratch_types=[
            pltpu.VMEM((R, CB), jnp.float32),
            pltpu.VMEM((R, CB), jnp.float32),
            pltpu.VMEM((CB,), jnp.int32),
            pltpu.SemaphoreType.DMA,
            pltpu.SemaphoreType.DMA,
        ],
    )(_sc_argmin)
    return kfn(x)


ROW_BLK = 512
N_K = N_ROW // ROW_BLK
N_COL_TC = N_COL - N_COL_SC


def _tc_argmin(x_ref, o_ref, mval, midx):
    k = pl.program_id(1)
    xb = x_ref[0]
    m = jnp.min(xb, axis=0, keepdims=True)
    rows = jax.lax.broadcasted_iota(jnp.int32, (ROW_BLK, N_COL_TC), 0) + k * ROW_BLK
    im = jnp.min(jnp.where(xb == m, rows, jnp.int32(2**30)), axis=0, keepdims=True)

    @pl.when(k == 0)
    def _init():
        mval[...] = m
        midx[...] = im

    @pl.when(k > 0)
    def _merge():
        better = m < mval[...]
        mval[...] = jnp.where(better, m, mval[...])
        midx[...] = jnp.where(better, im, midx[...])

    @pl.when(k == N_K - 1)
    def _emit():
        o_ref[0] = midx[...]


def _tc_call(x):
    out = pl.pallas_call(
        _tc_argmin,
        grid=(N_BATCH, N_K),
        in_specs=[
            pl.BlockSpec(
                (1, ROW_BLK, N_COL_TC),
                lambda b, k: (b, k, N_COL_SC // N_COL_TC),
            )
        ],
        out_specs=pl.BlockSpec((1, 1, N_COL_TC), lambda b, k: (b, 0, 0)),
        out_shape=jax.ShapeDtypeStruct((N_BATCH, 1, N_COL_TC), jnp.int32),
        scratch_shapes=[
            pltpu.VMEM((1, N_COL_TC), jnp.float32),
            pltpu.VMEM((1, N_COL_TC), jnp.int32),
        ],
    )(x)
    return out.reshape(N_BATCH, N_COL_TC)


def kernel(x):
    sc = _sc_call(x)
    tc = _tc_call(x)
    return jnp.concatenate([sc, tc], axis=1).astype(jnp.int64)


# speedup vs baseline: 1.8920x; 1.0017x over previous
import functools

import jax
import jax.numpy as jnp
from jax import lax
from jax.experimental import pallas as pl
from jax.experimental.pallas import tpu as pltpu
from jax.experimental.pallas import tpu_sc as plsc

N_BATCH = 4
N_ROW = 4096
N_COL = 2048
NC = 2
NS = 16
NW = NC * NS
L = 16
CB = 128
NCB = N_COL // CB
G = CB // L
N_BATCH_SC = 2
TASKS_W = N_BATCH_SC * NCB // NW
R = 256
N_CHUNK = N_ROW // R


def _sc_argmin(x_hbm, o_hbm, buf0, buf1, obuf, sem0, sem1):
    c = lax.axis_index("c")
    s = lax.axis_index("s")
    wid = s * NC + c
    bufs = (buf0, buf1)
    sems = (sem0, sem1)

    for t in range(TASKS_W):
        task = wid * TASKS_W + t
        b = task // NCB
        col0 = pl.multiple_of((task % NCB) * CB, CB)

        def chunk_src(k, _b=b, _col0=col0):
            return x_hbm.at[_b, pl.ds(k * R, R), pl.ds(_col0, CB)]

        pltpu.make_async_copy(chunk_src(0), bufs[0], sems[0]).start()
        mns = [jnp.full((L,), jnp.inf, jnp.float32) for _ in range(G)]
        mis = [jnp.full((L,), 0, jnp.int32) for _ in range(G)]
        for k in range(N_CHUNK):
            buf = bufs[k % 2]
            if k + 1 < N_CHUNK:
                pltpu.make_async_copy(
                    chunk_src(k + 1), bufs[(k + 1) % 2], sems[(k + 1) % 2]
                ).start()
            pltpu.make_async_copy(chunk_src(k), buf, sems[k % 2]).wait()

            def row_body(r4, carry, _buf=buf, _base=k * R):
                c_mns, c_mis = carry
                r0 = r4 * 4
                i0 = jnp.full((L,), 0, jnp.int32) + (_base + r0)
                i1 = i0 + 1
                i2 = i0 + 2
                i3 = i0 + 3
                n_mns, n_mis = [], []
                for g in range(G):
                    v0 = _buf[r0, pl.ds(g * L, L)]
                    v1 = _buf[r0 + 1, pl.ds(g * L, L)]
                    v2 = _buf[r0 + 2, pl.ds(g * L, L)]
                    v3 = _buf[r0 + 3, pl.ds(g * L, L)]
                    c01 = v1 < v0
                    m01 = jnp.minimum(v0, v1)
                    i01 = jnp.where(c01, i1, i0)
                    c23 = v3 < v2
                    m23 = jnp.minimum(v2, v3)
                    i23 = jnp.where(c23, i3, i2)
                    c03 = m23 < m01
                    m03 = jnp.minimum(m01, m23)
                    i03 = jnp.where(c03, i23, i01)
                    b = m03 < c_mns[g]
                    n_mns.append(jnp.where(b, m03, c_mns[g]))
                    n_mis.append(jnp.where(b, i03, c_mis[g]))
                return tuple(n_mns), tuple(n_mis)

            mns, mis = lax.fori_loop(
                0, R // 4, row_body, (tuple(mns), tuple(mis)), unroll=1
            )
            mns, mis = list(mns), list(mis)

        for g in range(G):
            obuf[pl.ds(g * L, L)] = mis[g]
        pltpu.sync_copy(obuf, o_hbm.at[b, pl.ds(col0, CB)])


def _sc_call(x):
    mesh = plsc.VectorSubcoreMesh(core_axis_name="c", subcore_axis_name="s")
    kfn = functools.partial(
        pl.kernel,
        mesh=mesh,
        out_type=jax.ShapeDtypeStruct((N_BATCH_SC, N_COL), jnp.int32),
        scratch_types=[
            pltpu.VMEM((R, CB), jnp.float32),
            pltpu.VMEM((R, CB), jnp.float32),
            pltpu.VMEM((CB,), jnp.int32),
            pltpu.SemaphoreType.DMA,
            pltpu.SemaphoreType.DMA,
        ],
    )(_sc_argmin)
    return kfn(x)


ROW_BLK = 512
N_K = N_ROW // ROW_BLK
N_BATCH_TC = N_BATCH - N_BATCH_SC


def _tc_argmin(x_ref, o_ref, mval, midx):
    k = pl.program_id(1)
    xb = x_ref[0]
    m = jnp.min(xb, axis=0, keepdims=True)
    rows = jax.lax.broadcasted_iota(jnp.int32, (ROW_BLK, N_COL), 0) + k * ROW_BLK
    im = jnp.min(jnp.where(xb == m, rows, jnp.int32(2**30)), axis=0, keepdims=True)

    @pl.when(k == 0)
    def _init():
        mval[...] = m
        midx[...] = im

    @pl.when(k > 0)
    def _merge():
        better = m < mval[...]
        mval[...] = jnp.where(better, m, mval[...])
        midx[...] = jnp.where(better, im, midx[...])

    @pl.when(k == N_K - 1)
    def _emit():
        o_ref[0] = midx[...]


def _tc_call(x):
    out = pl.pallas_call(
        _tc_argmin,
        grid=(N_BATCH_TC, N_K),
        in_specs=[
            pl.BlockSpec(
                (1, ROW_BLK, N_COL),
                lambda b, k: (b + N_BATCH_SC, k, 0),
            )
        ],
        out_specs=pl.BlockSpec((1, 1, N_COL), lambda b, k: (b, 0, 0)),
        out_shape=jax.ShapeDtypeStruct((N_BATCH_TC, 1, N_COL), jnp.int32),
        scratch_shapes=[
            pltpu.VMEM((1, N_COL), jnp.float32),
            pltpu.VMEM((1, N_COL), jnp.int32),
        ],
    )(x)
    return out.reshape(N_BATCH_TC, N_COL)


def kernel(x):
    sc = _sc_call(x)
    tc = _tc_call(x)
    return jnp.concatenate([sc, tc], axis=0).astype(jnp.int64)
